# baseline (speedup 1.0000x reference)
import functools

import jax
import jax.numpy as jnp
from jax import lax
from jax.experimental import pallas as pl
from jax.experimental.pallas import tpu as pltpu
from jax.experimental.pallas import tpu_sc as plsc

DIM = 32
LANES = 16
NC, NS = 2, 16
NW = NC * NS
NBUF = 2
CHUNK = 4
UNROLL = 8


def _sc_body(n_seq_per_w, seq, idx_hbm, tok_hbm, pos4_hbm, out_hbm,
             idx_v, rows_v, g0, g1, o0, o1, f0, f1):
    gsems = [g0, g1]
    osems = [o0, o1]
    fsems = [f0, f1]
    wid = lax.axis_index("s") * NC + lax.axis_index("c")
    w0 = wid * n_seq_per_w
    crows = CHUNK * seq

    for b in range(NBUF):
        pltpu.sync_copy(idx_hbm.at[pl.ds(w0 * seq + b * crows, crows)],
                        idx_v.at[b])
        pltpu.sync_copy(pos4_hbm, rows_v.at[b])
        pltpu.async_copy(tok_hbm.at[idx_v.at[b]], rows_v.at[b], gsems[b],
                         add=True)

    n_outer = n_seq_per_w // CHUNK // NBUF

    def outer(r, carry):
        for b in range(NBUF):
            s = r * NBUF + b
            base = w0 * seq + s * crows
            pltpu.make_async_copy(
                tok_hbm.at[idx_v.at[b]], rows_v.at[b], gsems[b]).wait()
            pltpu.async_copy(rows_v.at[b], out_hbm.at[pl.ds(base, crows)],
                             osems[b])

            @pl.when(r < n_outer - 1)
            def _(b=b, base=base):
                nbase = base + NBUF * crows
                pltpu.sync_copy(idx_hbm.at[pl.ds(nbase, crows)], idx_v.at[b])
                pltpu.make_async_copy(
                    rows_v.at[b], out_hbm.at[pl.ds(base, crows)],
                    osems[b]).wait()
                pltpu.async_copy(pos4_hbm, rows_v.at[b], fsems[b])
                pltpu.make_async_copy(pos4_hbm, rows_v.at[b],
                                      fsems[b]).wait()
                pltpu.async_copy(tok_hbm.at[idx_v.at[b]], rows_v.at[b],
                                 gsems[b], add=True)
        return carry

    lax.fori_loop(0, n_outer, outer, 0)
    for b in range(NBUF):
        pltpu.make_async_copy(rows_v.at[b], out_hbm.at[pl.ds(0, crows)],
                              osems[b]).wait()


def kernel(inputs, token_table, position_table):
    b, l = inputs.shape
    n = b * l
    flat_idx = inputs.reshape(n).astype(jnp.int32)
    n_seq_per_w = b // NW
    pos4 = jnp.tile(position_table, (CHUNK, 1))

    mesh = plsc.VectorSubcoreMesh(core_axis_name="c", subcore_axis_name="s",
                                  num_cores=NC, num_subcores=NS)
    out = pl.kernel(
        functools.partial(_sc_body, n_seq_per_w, l),
        out_type=jax.ShapeDtypeStruct((n, DIM), jnp.float32),
        mesh=mesh,
        scratch_types=[
            pltpu.VMEM((NBUF, CHUNK * l), jnp.int32),
            pltpu.VMEM((NBUF, CHUNK * l, DIM), jnp.float32),
            pltpu.SemaphoreType.DMA,
            pltpu.SemaphoreType.DMA,
            pltpu.SemaphoreType.DMA,
            pltpu.SemaphoreType.DMA,
            pltpu.SemaphoreType.DMA,
            pltpu.SemaphoreType.DMA,
        ],
        compiler_params=pltpu.CompilerParams(use_tc_tiling_on_sc=False),
    )(flat_idx, token_table, pos4)
    return out.reshape(b, l, DIM)


# speedup vs baseline: 1.1281x; 1.1281x over previous
import functools

import jax
import jax.numpy as jnp
from jax import lax
from jax.experimental import pallas as pl
from jax.experimental.pallas import tpu as pltpu
from jax.experimental.pallas import tpu_sc as plsc

DIM = 32
LANES = 16
NC, NS = 2, 16
NW = NC * NS
NBUF = 4
CHUNK = 2
UNROLL = 8


def _sc_body(n_seq_per_w, seq, idx_hbm, tok_hbm, pos_hbm, out_hbm,
             pos_v, idx_v, rows_v, *sems):
    gsems = sems[:NBUF]
    osems = sems[NBUF:]
    wid = lax.axis_index("s") * NC + lax.axis_index("c")
    w0 = wid * n_seq_per_w
    crows = CHUNK * seq
    pltpu.sync_copy(pos_hbm, pos_v)

    for b in range(NBUF):
        pltpu.sync_copy(idx_hbm.at[pl.ds(w0 * seq + b * crows, crows)],
                        idx_v.at[b])
        pltpu.async_copy(tok_hbm.at[idx_v.at[b]], rows_v.at[b], gsems[b])

    n_outer = n_seq_per_w // CHUNK // NBUF

    def outer(r, carry):
        for b in range(NBUF):
            s = r * NBUF + b
            base = w0 * seq + s * crows
            pltpu.make_async_copy(
                tok_hbm.at[idx_v.at[b]], rows_v.at[b], gsems[b]).wait()

            def add_rows(i, c, b=b):
                for dj in range(UNROLL):
                    row = i * UNROLL + dj
                    for j in range(DIM // LANES):
                        sl = pl.ds(j * LANES, LANES)
                        p = pos_v[row, sl]
                        for q in range(CHUNK):
                            rq = q * seq + row
                            rows_v[b, rq, sl] = rows_v[b, rq, sl] + p
                return c

            lax.fori_loop(0, seq // UNROLL, add_rows, 0)
            pltpu.async_copy(rows_v.at[b], out_hbm.at[pl.ds(base, crows)],
                             osems[b])

            @pl.when(r < n_outer - 1)
            def _(b=b, s=s, base=base):
                nbase = base + NBUF * crows
                pltpu.sync_copy(idx_hbm.at[pl.ds(nbase, crows)], idx_v.at[b])
                pltpu.make_async_copy(
                    rows_v.at[b], out_hbm.at[pl.ds(base, crows)],
                    osems[b]).wait()
                pltpu.async_copy(tok_hbm.at[idx_v.at[b]], rows_v.at[b],
                                 gsems[b])
        return carry

    lax.fori_loop(0, n_outer, outer, 0)
    for b in range(NBUF):
        pltpu.make_async_copy(rows_v.at[b], out_hbm.at[pl.ds(0, crows)],
                              osems[b]).wait()


def kernel(inputs, token_table, position_table):
    b, l = inputs.shape
    n = b * l
    flat_idx = inputs.reshape(n).astype(jnp.int32)
    n_seq_per_w = b // NW

    mesh = plsc.VectorSubcoreMesh(core_axis_name="c", subcore_axis_name="s",
                                  num_cores=NC, num_subcores=NS)
    out = pl.kernel(
        functools.partial(_sc_body, n_seq_per_w, l),
        out_type=jax.ShapeDtypeStruct((n, DIM), jnp.float32),
        mesh=mesh,
        scratch_types=[
            pltpu.VMEM((l, DIM), jnp.float32),
            pltpu.VMEM((NBUF, CHUNK * l), jnp.int32),
            pltpu.VMEM((NBUF, CHUNK * l, DIM), jnp.float32),
        ] + [pltpu.SemaphoreType.DMA] * (2 * NBUF),
        compiler_params=pltpu.CompilerParams(use_tc_tiling_on_sc=False),
    )(flat_idx, token_table, position_table)
    return out.reshape(b, l, DIM)


# speedup vs baseline: 1.1477x; 1.0174x over previous
import functools

import jax
import jax.numpy as jnp
from jax import lax
from jax.experimental import pallas as pl
from jax.experimental.pallas import tpu as pltpu
from jax.experimental.pallas import tpu_sc as plsc

DIM = 32
LANES = 16
NC, NS = 2, 16
NW = NC * NS
NBUF = 2
CHUNK = 8
UNROLL = 8


def _sc_body(n_seq_per_w, seq, idx_hbm, tok_hbm, pos_hbm, out_hbm,
             pos_v, idx_v, rows_v, *sems):
    gsems = sems[:NBUF]
    osems = sems[NBUF:]
    wid = lax.axis_index("s") * NC + lax.axis_index("c")
    w0 = wid * n_seq_per_w
    crows = CHUNK * seq
    pltpu.sync_copy(pos_hbm, pos_v)

    for b in range(NBUF):
        pltpu.sync_copy(idx_hbm.at[pl.ds(w0 * seq + b * crows, crows)],
                        idx_v.at[b])
        pltpu.async_copy(tok_hbm.at[idx_v.at[b]], rows_v.at[b], gsems[b])

    n_outer = n_seq_per_w // CHUNK // NBUF

    def outer(r, carry):
        for b in range(NBUF):
            s = r * NBUF + b
            base = w0 * seq + s * crows
            pltpu.make_async_copy(
                tok_hbm.at[idx_v.at[b]], rows_v.at[b], gsems[b]).wait()

            def add_rows(i, c, b=b):
                for dj in range(UNROLL):
                    row = i * UNROLL + dj
                    for j in range(DIM // LANES):
                        sl = pl.ds(j * LANES, LANES)
                        p = pos_v[row, sl]
                        for q in range(CHUNK):
                            rq = q * seq + row
                            rows_v[b, rq, sl] = rows_v[b, rq, sl] + p
                return c

            lax.fori_loop(0, seq // UNROLL, add_rows, 0)
            pltpu.async_copy(rows_v.at[b], out_hbm.at[pl.ds(base, crows)],
                             osems[b])

            @pl.when(r < n_outer - 1)
            def _(b=b, s=s, base=base):
                nbase = base + NBUF * crows
                pltpu.sync_copy(idx_hbm.at[pl.ds(nbase, crows)], idx_v.at[b])
                pltpu.make_async_copy(
                    rows_v.at[b], out_hbm.at[pl.ds(base, crows)],
                    osems[b]).wait()
                pltpu.async_copy(tok_hbm.at[idx_v.at[b]], rows_v.at[b],
                                 gsems[b])
        return carry

    lax.fori_loop(0, n_outer, outer, 0)
    for b in range(NBUF):
        pltpu.make_async_copy(rows_v.at[b], out_hbm.at[pl.ds(0, crows)],
                              osems[b]).wait()


def kernel(inputs, token_table, position_table):
    b, l = inputs.shape
    n = b * l
    flat_idx = inputs.reshape(n).astype(jnp.int32)
    n_seq_per_w = b // NW

    mesh = plsc.VectorSubcoreMesh(core_axis_name="c", subcore_axis_name="s",
                                  num_cores=NC, num_subcores=NS)
    out = pl.kernel(
        functools.partial(_sc_body, n_seq_per_w, l),
        out_type=jax.ShapeDtypeStruct((n, DIM), jnp.float32),
        mesh=mesh,
        scratch_types=[
            pltpu.VMEM((l, DIM), jnp.float32),
            pltpu.VMEM((NBUF, CHUNK * l), jnp.int32),
            pltpu.VMEM((NBUF, CHUNK * l, DIM), jnp.float32),
        ] + [pltpu.SemaphoreType.DMA] * (2 * NBUF),
        compiler_params=pltpu.CompilerParams(use_tc_tiling_on_sc=False),
    )(flat_idx, token_table, position_table)
    return out.reshape(b, l, DIM)
